# Initial kernel scaffold; baseline (speedup 1.0000x reference)
import jax
import jax.numpy as jnp
from jax.experimental import pallas as pl


def kernel(action_emb, item_embs, recommended_ids):
    raise NotImplementedError("write your pallas kernel here")


# speedup vs baseline: 7.1058x; 7.1058x over previous
import functools

import jax
import jax.numpy as jnp
from jax import lax
from jax.experimental import pallas as pl
from jax.experimental.pallas import tpu as pltpu
from jax.experimental.pallas import tpu_sc as plsc

B = 1024
N_ITEMS = 100000
EMB_DIM = 64
SLATE = 10

TN = 2048
N_PAD = 100352
GRID_N = N_PAD // TN

NW = 32
RPW = B // NW
REC_PAD = 64
CAP = 2048
CH = 512
NCH = N_PAD // CH
NEG = -3.0e38
MASKVAL = -1e9


def _score_body(a_ref, it_ref, o_ref):
    it = it_ref[...]
    norm = jnp.sqrt(jnp.sum(it * it, axis=1, keepdims=True))
    itn = it / jnp.maximum(norm, 1e-12)
    s = lax.dot_general(a_ref[...], itn, (((1,), (1,)), ((), ())),
                        preferred_element_type=jnp.float32)
    j = pl.program_id(0)
    col = j * TN + lax.broadcasted_iota(jnp.int32, (1, TN), 1)
    o_ref[...] = jnp.where(col < N_ITEMS, s, MASKVAL)


def _scores_tc(action_emb, items_padded):
    return pl.pallas_call(
        _score_body,
        grid=(GRID_N,),
        in_specs=[
            pl.BlockSpec((B, EMB_DIM), lambda j: (0, 0)),
            pl.BlockSpec((TN, EMB_DIM), lambda j: (j, 0)),
        ],
        out_specs=pl.BlockSpec((B, TN), lambda j: (0, j)),
        out_shape=jax.ShapeDtypeStruct((B, N_PAD), jnp.float32),
    )(action_emb, items_padded)


def _topk_body(scores_hbm, rec_hbm, ov_hbm, oi_hbm,
               row_v, rec_v, cmax_v, cv_v, ci_v, tv_v, ti_v):
    wid = lax.axis_index("s") * 2 + lax.axis_index("c")
    iota = lax.iota(jnp.int32, 16)
    negv = jnp.full((16,), NEG, jnp.float32)

    def row_body(r, carry):
        row = wid * RPW + r
        pltpu.sync_copy(scores_hbm.at[pl.ds(pl.multiple_of(row * N_PAD, 8), N_PAD)], row_v)
        pltpu.sync_copy(rec_hbm.at[pl.ds(pl.multiple_of(row * REC_PAD, 8), REC_PAD)], rec_v)

        for h in range(REC_PAD // 16):
            idxv = rec_v[pl.ds(h * 16, 16)]
            plsc.store_scatter(row_v, [idxv], jnp.full((16,), MASKVAL, jnp.float32))

        def chunk_a(c, gacc):
            acc = negv
            for j in range(CH // 16):
                acc = jnp.maximum(acc, row_v[pl.ds(c * CH + j * 16, 16)])
            cmax_v[pl.ds(c * 16, 16)] = acc
            return jnp.maximum(gacc, acc)

        gacc = lax.fori_loop(0, NCH, chunk_a, negv)

        srt = lax.sort(gacc)
        t = jnp.max(jnp.where(iota == 6, srt, NEG))
        t_vec = jnp.full((16,), t)

        def chunk_b(c, cnt_vec):
            cm = cmax_v[pl.ds(c * 16, 16)]
            s = jnp.max(cm)

            def live(cv):
                def vbody(j, cv):
                    base = c * CH + j * 16
                    v = row_v[pl.ds(base, 16)]
                    m = v >= t_vec
                    pc = plsc.all_reduce_population_count(m)
                    ps = plsc.cumsum(m.astype(jnp.int32))
                    pos = cv + ps - 1
                    wm = jnp.logical_and(m, pos < CAP)
                    pos = jnp.clip(pos, 0, CAP - 1)
                    plsc.store_scatter(cv_v, [pos], v, mask=wm)
                    plsc.store_scatter(ci_v, [pos], base + iota, mask=wm)
                    return cv + pc

                return lax.fori_loop(0, CH // 16, vbody, cv)

            return lax.cond(s >= t, live, lambda cv: cv, cnt_vec)

        cnt_vec = lax.fori_loop(0, NCH, chunk_b, jnp.zeros((16,), jnp.int32))
        count = jnp.minimum(jnp.max(cnt_vec), CAP)
        c_vec = jnp.full((16,), count)
        nv = (count + 15) // 16

        def round_body(k, st):
            resv, resi = st

            def fold(j, best):
                v = cv_v[pl.ds(j * 16, 16)]
                v = jnp.where(j * 16 + iota < c_vec, v, NEG)
                return jnp.maximum(best, v)

            best = lax.fori_loop(0, nv, fold, negv)
            mval = jnp.max(best)
            mvec = jnp.full((16,), mval)

            def find(j, fpos):
                v = cv_v[pl.ds(j * 16, 16)]
                gpos = j * 16 + iota
                eq = jnp.logical_and(v == mvec, gpos < c_vec)
                return jnp.minimum(fpos, jnp.min(jnp.where(eq, gpos, CAP)))

            fpos = lax.fori_loop(0, nv, find, CAP)
            pos_vec = jnp.full((16,), fpos)
            iv = plsc.load_gather(ci_v, [pos_vec])
            resv = jnp.where(iota == k, mvec, resv)
            resi = jnp.where(iota == k, iv, resi)
            plsc.store_scatter(cv_v, [pos_vec], negv, mask=iota == 0)
            return (resv, resi)

        resv, resi = lax.fori_loop(0, SLATE, round_body,
                                   (negv, jnp.zeros((16,), jnp.int32)))

        tv_v[...] = resv
        ti_v[...] = resi
        pltpu.sync_copy(tv_v, ov_hbm.at[pl.ds(pl.multiple_of(row * 16, 8), 16)])
        pltpu.sync_copy(ti_v, oi_hbm.at[pl.ds(pl.multiple_of(row * 16, 8), 16)])
        return carry

    lax.fori_loop(0, RPW, row_body, 0)


_topk_sc = functools.partial(
    pl.kernel,
    out_type=(jax.ShapeDtypeStruct((B * 16,), jnp.float32),
              jax.ShapeDtypeStruct((B * 16,), jnp.int32)),
    mesh=plsc.VectorSubcoreMesh(core_axis_name="c", subcore_axis_name="s"),
    compiler_params=pltpu.CompilerParams(needs_layout_passes=False),
    scratch_types=[
        pltpu.VMEM((N_PAD,), jnp.float32),
        pltpu.VMEM((REC_PAD,), jnp.int32),
        pltpu.VMEM((NCH * 16,), jnp.float32),
        pltpu.VMEM((CAP,), jnp.float32),
        pltpu.VMEM((CAP,), jnp.int32),
        pltpu.VMEM((16,), jnp.float32),
        pltpu.VMEM((16,), jnp.int32),
    ],
)(_topk_body)


def kernel(action_emb, item_embs, recommended_ids):
    items_padded = jnp.pad(item_embs, ((0, N_PAD - N_ITEMS), (0, 0)))
    scores = _scores_tc(action_emb, items_padded)
    rec = recommended_ids.astype(jnp.int32)
    recp = jnp.pad(rec, ((0, 0), (0, REC_PAD - rec.shape[1])),
                   constant_values=N_ITEMS)
    ov, oi = _topk_sc(scores.reshape(-1), recp.reshape(-1))
    return ov.reshape(B, 16)[:, :SLATE], oi.reshape(B, 16)[:, :SLATE]
